# baseline (speedup 1.0000x reference)
import functools

import numpy as np
import jax
import jax.numpy as jnp
from jax import lax
from jax.experimental import pallas as pl
from jax.experimental.pallas import tpu as pltpu
from jax.experimental.pallas import tpu_sc as plsc

L_LEVELS = 16
F_FEAT = 2
T_SIZE = 2 ** 19
B_PTS = 262144
APP_DIM = 27

P2 = np.int32(-1640531535)
P3 = np.int32(805459861)

NW = 32
PTS_PER_W = B_PTS // NW
BLK = 1024
NCHUNK = BLK // 16
NBLOCKS = PTS_PER_W // BLK
PIPE = 4

DL = 5
D_G = [3, 5, 9, 17, 33]
D_OFF = [0, 27, 152, 881, 5794]
D_TOT = 41731
NSTREAM = L_LEVELS - DL
CW = NSTREAM * 128
D_PAD = 8 * (4 * CW)

def _dense_idx():
    parts = []
    for l in range(DL):
        g = np.uint32(D_G[l])
        xs = np.arange(g, dtype=np.uint32)
        hy = xs * np.uint32(2654435761)
        hz = xs * np.uint32(805459861)
        h = (xs[:, None, None] ^ hy[None, :, None] ^ hz[None, None, :]) \
            & np.uint32(T_SIZE - 1)
        parts.append((np.int64(l * T_SIZE) + h.reshape(-1)).astype(np.int32))
    flat = np.concatenate(parts)
    return np.concatenate([flat, np.zeros(D_PAD - flat.size, np.int32)])

DIDX = _dense_idx()

_MASK = np.int32(T_SIZE - 1)


def _encode_body(cx_hbm, cy_hbm, cz_hbm, cd_hbm, tpk_hbm, didx_hbm, out_hbm,
                 cx_v, cy_v, cz_v, cd_v, idx_v, rows_v, feat_v, dense_v,
                 gsem):
    nc = 2
    wid = lax.axis_index("s") * nc + lax.axis_index("c")
    lanes = lax.iota(jnp.int32, 16)

    def load_xyz(o):
        invd = cd_v[pl.ds(o, 16)]
        return (cx_v[pl.ds(o, 16)] * invd,
                cy_v[pl.ds(o, 16)] * invd,
                cz_v[pl.ds(o, 16)] * invd)

    def fire(ci, par):
        x0, x1, x2 = load_xyz(ci * 16)
        for l in range(DL, L_LEVELS):
            res = np.float32(2.0 * (2.0 ** l))
            px = x0 * res
            py = x1 * res
            pz = x2 * res
            ix = px.astype(jnp.int32)
            iy = py.astype(jnp.int32)
            iz = pz.astype(jnp.int32)
            hx0 = ix
            hx1 = ix + np.int32(1)
            hy0 = iy * P2
            hy1 = hy0 + P2
            hz0 = iz * P3
            hz1 = hz0 + P3
            base_l = np.int32(l * T_SIZE)
            corner = 0
            for hx in (hx0, hx1):
                for hy in (hy0, hy1):
                    for hz in (hz0, hz1):
                        idx = ((hx ^ hy ^ hz) & _MASK) + base_l
                        idx_v[pl.ds(par * np.int32(CW) + np.int32((l - DL) * 128 + corner * 16), 16)] = idx
                        corner += 1
        pltpu.async_copy(tpk_hbm.at[idx_v.at[pl.ds(par * np.int32(CW), CW)]],
                         rows_v.at[pl.ds(par * np.int32(CW), CW)], gsem)

    def drain_and_interp(ci, par):
        pltpu.make_async_copy(tpk_hbm.at[idx_v.at[pl.ds(par * np.int32(CW), CW)]],
                              rows_v.at[pl.ds(par * np.int32(CW), CW)],
                              gsem).wait()

        o = ci * 16
        x0, x1, x2 = load_xyz(o)
        pbase_i = (o + lanes) * np.int32(2 * L_LEVELS)
        for l in range(L_LEVELS):
            res = np.float32(2.0 * (2.0 ** l))
            px = x0 * res
            py = x1 * res
            pz = x2 * res
            ix = px.astype(jnp.int32)
            iy = py.astype(jnp.int32)
            iz = pz.astype(jnp.int32)
            w0 = px - ix.astype(jnp.float32)
            w1 = py - iy.astype(jnp.float32)
            w2 = pz - iz.astype(jnp.float32)
            u0 = np.float32(1.0) - w0
            u1 = np.float32(1.0) - w1
            u2 = np.float32(1.0) - w2
            acc0 = jnp.zeros((16,), jnp.float32)
            acc1 = jnp.zeros((16,), jnp.float32)
            if l < DL:
                g = D_G[l]
                gg = g * g
                e0 = ix * np.int32(gg) + iy * np.int32(g) + iz + np.int32(D_OFF[l])
            else:
                row = par * np.int32(CW) + np.int32((l - DL) * 128)
            corner = 0
            for ci_x, wxv in ((0, u0), (1, w0)):
                for ci_y, wyv in ((0, u1), (1, w1)):
                    for ci_z, wzv in ((0, u2), (1, w2)):
                        ww = (wxv * wyv) * wzv
                        if l < DL:
                            off = ci_x * gg + ci_y * g + ci_z
                            v = plsc.load_gather(dense_v, [e0 + np.int32(off)])
                        else:
                            v = rows_v[pl.ds(row + np.int32(corner * 16), 16)]
                        f0 = plsc.bitcast(v & np.int32(-65536), jnp.float32)
                        f1 = plsc.bitcast(v << np.int32(16), jnp.float32)
                        acc0 = acc0 + f0 * ww
                        acc1 = acc1 + f1 * ww
                        corner += 1
            plsc.store_scatter(feat_v, [pbase_i + np.int32(2 * l)], acc0)
            plsc.store_scatter(feat_v, [pbase_i + np.int32(2 * l + 1)], acc1)

    PC = PIPE * CW
    for k in range(D_PAD // PC):
        pltpu.sync_copy(didx_hbm.at[pl.ds(k * PC, PC)], idx_v)
        pltpu.async_copy(tpk_hbm.at[idx_v.at[pl.ds(0, PC)]],
                         dense_v.at[pl.ds(k * PC, PC)], gsem)
        pltpu.make_async_copy(tpk_hbm.at[idx_v.at[pl.ds(0, PC)]],
                              dense_v.at[pl.ds(k * PC, PC)], gsem).wait()

    def block_body(b, carry):
        pbase = wid * PTS_PER_W + b * BLK
        pltpu.sync_copy(cx_hbm.at[pl.ds(pbase, BLK)], cx_v)
        pltpu.sync_copy(cy_hbm.at[pl.ds(pbase, BLK)], cy_v)
        pltpu.sync_copy(cz_hbm.at[pl.ds(pbase, BLK)], cz_v)
        pltpu.sync_copy(cd_hbm.at[pl.ds(pbase, BLK)], cd_v)

        def chunk_body(ci, carry2):
            par = lax.rem(ci, np.int32(PIPE))

            @pl.when(ci >= PIPE)
            def _():
                drain_and_interp(ci - PIPE, par)

            @pl.when(ci < NCHUNK)
            def _():
                fire(ci, par)

            return carry2

        lax.fori_loop(0, NCHUNK + PIPE, chunk_body, 0)
        pltpu.sync_copy(feat_v, out_hbm.at[pl.ds(pbase * np.int32(2 * L_LEVELS),
                                                 BLK * 2 * L_LEVELS)])
        return carry

    lax.fori_loop(0, NBLOCKS, block_body, 0)


@jax.jit
def _encode(cx, cy, cz, cd, tpk, didx):
    mesh = plsc.VectorSubcoreMesh(core_axis_name="c", subcore_axis_name="s")
    fn = functools.partial(
        pl.kernel,
        mesh=mesh,
        compiler_params=pltpu.CompilerParams(needs_layout_passes=False),
        out_type=jax.ShapeDtypeStruct((B_PTS * 2 * L_LEVELS,), jnp.float32),
        scratch_types=[
            pltpu.VMEM((BLK,), jnp.float32),
            pltpu.VMEM((BLK,), jnp.float32),
            pltpu.VMEM((BLK,), jnp.float32),
            pltpu.VMEM((BLK,), jnp.float32),
            pltpu.VMEM((PIPE * CW,), jnp.int32),
            pltpu.VMEM((PIPE * CW,), jnp.int32),
            pltpu.VMEM((BLK * 2 * L_LEVELS,), jnp.float32),
            pltpu.VMEM((D_PAD,), jnp.int32),
            pltpu.SemaphoreType.DMA,
        ],
    )(_encode_body)
    return fn(cx, cy, cz, cd, tpk, didx).reshape(B_PTS, 2 * L_LEVELS)


def _mlp_body(feat_ref, w1_ref, w2_ref, w3_ref, out_ref):
    f = feat_ref[...]
    h = jnp.maximum(jnp.dot(f, w1_ref[...], preferred_element_type=jnp.float32), 0.0)
    h = jnp.maximum(jnp.dot(h, w2_ref[...], preferred_element_type=jnp.float32), 0.0)
    out_ref[...] = jnp.dot(h, w3_ref[...], preferred_element_type=jnp.float32)


@jax.jit
def _mlp(feats, W1, W2, W3):
    BM = 2048
    return pl.pallas_call(
        _mlp_body,
        grid=(B_PTS // BM,),
        in_specs=[
            pl.BlockSpec((BM, 2 * L_LEVELS), lambda i: (i, 0)),
            pl.BlockSpec((2 * L_LEVELS, 128), lambda i: (0, 0)),
            pl.BlockSpec((128, 128), lambda i: (0, 0)),
            pl.BlockSpec((128, 32), lambda i: (0, 0)),
        ],
        out_specs=pl.BlockSpec((BM, 32), lambda i: (i, 0)),
        out_shape=jax.ShapeDtypeStruct((B_PTS, 32), jnp.float32),
    )(feats, W1, W2, W3)


def kernel(xyz_env_normed, table, W1d, W2d, W3d, W1r, W2r, W3r):
    cx = xyz_env_normed[:, 0]
    cy = xyz_env_normed[:, 1]
    cz = xyz_env_normed[:, 2]
    cd = xyz_env_normed[:, 3]
    bits = jax.lax.bitcast_convert_type(
        table.astype(jnp.bfloat16), jnp.uint16).astype(jnp.uint32)
    tpk = jax.lax.bitcast_convert_type(
        (bits[:, :, 0] << jnp.uint32(16)) | bits[:, :, 1],
        jnp.int32).reshape(L_LEVELS * T_SIZE)

    feats = _encode(cx, cy, cz, cd, tpk, jnp.asarray(DIDX))

    Z = jnp.zeros((64, 64), jnp.float32)
    W1 = jnp.concatenate([W1d, W1r], axis=1)
    W2 = jnp.concatenate(
        [jnp.concatenate([W2d, Z], axis=1),
         jnp.concatenate([Z, W2r], axis=1)], axis=0)
    W3 = jnp.zeros((128, 32), jnp.float32)
    W3 = W3.at[:64, 0:1].set(W3d)
    W3 = W3.at[64:, 1:1 + APP_DIM].set(W3r)

    out = _mlp(feats, W1, W2, W3)
    sigma = out[:, 0]
    app_feat = out[:, 1:1 + APP_DIM]
    return (sigma, app_feat)


# speedup vs baseline: 1.0037x; 1.0037x over previous
import functools

import numpy as np
import jax
import jax.numpy as jnp
from jax import lax
from jax.experimental import pallas as pl
from jax.experimental.pallas import tpu as pltpu
from jax.experimental.pallas import tpu_sc as plsc

L_LEVELS = 16
F_FEAT = 2
T_SIZE = 2 ** 19
B_PTS = 262144
APP_DIM = 27

P2 = np.int32(-1640531535)
P3 = np.int32(805459861)

NW = 32
PTS_PER_W = B_PTS // NW
BLK = 1024
NCHUNK = BLK // 16
NBLOCKS = PTS_PER_W // BLK
PIPE = 4

DL = 5
D_G = [3, 5, 9, 17, 33]
D_OFF = [0, 27, 152, 881, 5794]
D_TOT = 41731
NSTREAM = L_LEVELS - DL
CW = NSTREAM * 128
D_PAD = 8 * (4 * CW)

def _dense_idx():
    parts = []
    for l in range(DL):
        g = np.uint32(D_G[l])
        xs = np.arange(g, dtype=np.uint32)
        hy = xs * np.uint32(2654435761)
        hz = xs * np.uint32(805459861)
        h = (xs[:, None, None] ^ hy[None, :, None] ^ hz[None, None, :]) \
            & np.uint32(T_SIZE - 1)
        parts.append((np.int64(l * T_SIZE) + h.reshape(-1)).astype(np.int32))
    flat = np.concatenate(parts)
    return np.concatenate([flat, np.zeros(D_PAD - flat.size, np.int32)])

DIDX = _dense_idx()

_MASK = np.int32(T_SIZE - 1)


def _encode_body(cx_hbm, cy_hbm, cz_hbm, cd_hbm, tpk_hbm, didx_hbm, out_hbm,
                 cx_v, cy_v, cz_v, cd_v, idx_v, rows_v, feat_v, dense_v,
                 gsem):
    nc = 2
    wid = lax.axis_index("s") * nc + lax.axis_index("c")
    lanes = lax.iota(jnp.int32, 16)

    def load_xyz(o):
        invd = cd_v[pl.ds(o, 16)]
        return (cx_v[pl.ds(o, 16)] * invd,
                cy_v[pl.ds(o, 16)] * invd,
                cz_v[pl.ds(o, 16)] * invd)

    def fire(ci, par):
        x0, x1, x2 = load_xyz(ci * 16)
        for l in range(DL, L_LEVELS):
            res = np.float32(2.0 * (2.0 ** l))
            px = x0 * res
            py = x1 * res
            pz = x2 * res
            ix = px.astype(jnp.int32)
            iy = py.astype(jnp.int32)
            iz = pz.astype(jnp.int32)
            hx0 = ix
            hx1 = ix + np.int32(1)
            hy0 = iy * P2
            hy1 = hy0 + P2
            hz0 = iz * P3
            hz1 = hz0 + P3
            base_l = np.int32(l * T_SIZE)
            corner = 0
            for hx in (hx0, hx1):
                for hy in (hy0, hy1):
                    for hz in (hz0, hz1):
                        idx = ((hx ^ hy ^ hz) & _MASK) + base_l
                        idx_v[pl.ds(par * np.int32(CW) + np.int32((l - DL) * 128 + corner * 16), 16)] = idx
                        corner += 1
        pltpu.async_copy(tpk_hbm.at[idx_v.at[pl.ds(par * np.int32(CW), CW)]],
                         rows_v.at[pl.ds(par * np.int32(CW), CW)], gsem)

    def drain_and_interp(ci, par):
        pltpu.make_async_copy(tpk_hbm.at[idx_v.at[pl.ds(par * np.int32(CW), CW)]],
                              rows_v.at[pl.ds(par * np.int32(CW), CW)],
                              gsem).wait()

        o = ci * 16
        x0, x1, x2 = load_xyz(o)
        pbase_i = (o + lanes) * np.int32(2 * L_LEVELS)
        for l in range(L_LEVELS):
            res = np.float32(2.0 * (2.0 ** l))
            px = x0 * res
            py = x1 * res
            pz = x2 * res
            ix = px.astype(jnp.int32)
            iy = py.astype(jnp.int32)
            iz = pz.astype(jnp.int32)
            w0 = px - ix.astype(jnp.float32)
            w1 = py - iy.astype(jnp.float32)
            w2 = pz - iz.astype(jnp.float32)
            u0 = np.float32(1.0) - w0
            u1 = np.float32(1.0) - w1
            u2 = np.float32(1.0) - w2
            acc0 = jnp.zeros((16,), jnp.float32)
            acc1 = jnp.zeros((16,), jnp.float32)
            if l < DL:
                g = D_G[l]
                gg = g * g
                e0 = ix * np.int32(gg) + iy * np.int32(g) + iz + np.int32(D_OFF[l])
            else:
                row = par * np.int32(CW) + np.int32((l - DL) * 128)
            corner = 0
            for ci_x, wxv in ((0, u0), (1, w0)):
                for ci_y, wyv in ((0, u1), (1, w1)):
                    for ci_z, wzv in ((0, u2), (1, w2)):
                        ww = (wxv * wyv) * wzv
                        if l < DL:
                            off = ci_x * gg + ci_y * g + ci_z
                            v = plsc.load_gather(dense_v, [e0 + np.int32(off)])
                        else:
                            v = rows_v[pl.ds(row + np.int32(corner * 16), 16)]
                        f0 = plsc.bitcast(v & np.int32(-65536), jnp.float32)
                        f1 = plsc.bitcast(v << np.int32(16), jnp.float32)
                        acc0 = acc0 + f0 * ww
                        acc1 = acc1 + f1 * ww
                        corner += 1
            plsc.store_scatter(feat_v, [pbase_i + np.int32(2 * l)], acc0)
            plsc.store_scatter(feat_v, [pbase_i + np.int32(2 * l + 1)], acc1)

    PC = (PIPE * CW) // 2
    NP = D_PAD // PC

    def _dense_copy(k, half):
        return pltpu.make_async_copy(
            tpk_hbm.at[idx_v.at[pl.ds(half * PC, PC)]],
            dense_v.at[pl.ds(k * PC, PC)], gsem)

    for k in range(NP):
        half = k % 2
        if k >= 2:
            _dense_copy(k - 2, half).wait()
        pltpu.sync_copy(didx_hbm.at[pl.ds(k * PC, PC)],
                        idx_v.at[pl.ds(half * PC, PC)])
        _dense_copy(k, half).start()
    _dense_copy(NP - 2, 0).wait()
    _dense_copy(NP - 1, 1).wait()

    def block_body(b, carry):
        pbase = wid * PTS_PER_W + b * BLK
        pltpu.sync_copy(cx_hbm.at[pl.ds(pbase, BLK)], cx_v)
        pltpu.sync_copy(cy_hbm.at[pl.ds(pbase, BLK)], cy_v)
        pltpu.sync_copy(cz_hbm.at[pl.ds(pbase, BLK)], cz_v)
        pltpu.sync_copy(cd_hbm.at[pl.ds(pbase, BLK)], cd_v)

        def chunk_body(ci, carry2):
            par = lax.rem(ci, np.int32(PIPE))

            @pl.when(ci >= PIPE)
            def _():
                drain_and_interp(ci - PIPE, par)

            @pl.when(ci < NCHUNK)
            def _():
                fire(ci, par)

            return carry2

        lax.fori_loop(0, NCHUNK + PIPE, chunk_body, 0)
        pltpu.sync_copy(feat_v, out_hbm.at[pl.ds(pbase * np.int32(2 * L_LEVELS),
                                                 BLK * 2 * L_LEVELS)])
        return carry

    lax.fori_loop(0, NBLOCKS, block_body, 0)


@jax.jit
def _encode(cx, cy, cz, cd, tpk, didx):
    mesh = plsc.VectorSubcoreMesh(core_axis_name="c", subcore_axis_name="s")
    fn = functools.partial(
        pl.kernel,
        mesh=mesh,
        compiler_params=pltpu.CompilerParams(needs_layout_passes=False),
        out_type=jax.ShapeDtypeStruct((B_PTS * 2 * L_LEVELS,), jnp.float32),
        scratch_types=[
            pltpu.VMEM((BLK,), jnp.float32),
            pltpu.VMEM((BLK,), jnp.float32),
            pltpu.VMEM((BLK,), jnp.float32),
            pltpu.VMEM((BLK,), jnp.float32),
            pltpu.VMEM((PIPE * CW,), jnp.int32),
            pltpu.VMEM((PIPE * CW,), jnp.int32),
            pltpu.VMEM((BLK * 2 * L_LEVELS,), jnp.float32),
            pltpu.VMEM((D_PAD,), jnp.int32),
            pltpu.SemaphoreType.DMA,
        ],
    )(_encode_body)
    return fn(cx, cy, cz, cd, tpk, didx).reshape(B_PTS, 2 * L_LEVELS)


def _mlp_body(feat_ref, w1_ref, w2_ref, w3_ref, sig_ref, app_ref):
    f = feat_ref[...]
    h = jnp.maximum(jnp.dot(f, w1_ref[...], preferred_element_type=jnp.float32), 0.0)
    h = jnp.maximum(jnp.dot(h, w2_ref[...], preferred_element_type=jnp.float32), 0.0)
    out = jnp.dot(h, w3_ref[...], preferred_element_type=jnp.float32)
    sig_ref[...] = out[:, :1]
    app_ref[...] = out[:, 1:1 + APP_DIM]


@jax.jit
def _mlp(feats, W1, W2, W3):
    BM = 2048
    return pl.pallas_call(
        _mlp_body,
        grid=(B_PTS // BM,),
        in_specs=[
            pl.BlockSpec((BM, 2 * L_LEVELS), lambda i: (i, 0)),
            pl.BlockSpec((2 * L_LEVELS, 128), lambda i: (0, 0)),
            pl.BlockSpec((128, 128), lambda i: (0, 0)),
            pl.BlockSpec((128, 32), lambda i: (0, 0)),
        ],
        out_specs=[
            pl.BlockSpec((BM, 1), lambda i: (i, 0)),
            pl.BlockSpec((BM, APP_DIM), lambda i: (i, 0)),
        ],
        out_shape=[
            jax.ShapeDtypeStruct((B_PTS, 1), jnp.float32),
            jax.ShapeDtypeStruct((B_PTS, APP_DIM), jnp.float32),
        ],
    )(feats, W1, W2, W3)


def kernel(xyz_env_normed, table, W1d, W2d, W3d, W1r, W2r, W3r):
    cx = xyz_env_normed[:, 0]
    cy = xyz_env_normed[:, 1]
    cz = xyz_env_normed[:, 2]
    cd = xyz_env_normed[:, 3]
    bits = jax.lax.bitcast_convert_type(
        table.astype(jnp.bfloat16), jnp.uint16).astype(jnp.uint32)
    tpk = jax.lax.bitcast_convert_type(
        (bits[:, :, 0] << jnp.uint32(16)) | bits[:, :, 1],
        jnp.int32).reshape(L_LEVELS * T_SIZE)

    feats = _encode(cx, cy, cz, cd, tpk, jnp.asarray(DIDX))

    Z = jnp.zeros((64, 64), jnp.float32)
    W1 = jnp.concatenate([W1d, W1r], axis=1)
    W2 = jnp.concatenate(
        [jnp.concatenate([W2d, Z], axis=1),
         jnp.concatenate([Z, W2r], axis=1)], axis=0)
    W3 = jnp.zeros((128, 32), jnp.float32)
    W3 = W3.at[:64, 0:1].set(W3d)
    W3 = W3.at[64:, 1:1 + APP_DIM].set(W3r)

    sig, app_feat = _mlp(feats, W1, W2, W3)
    return (sig[:, 0], app_feat)


# speedup vs baseline: 1.3611x; 1.3561x over previous
import functools

import numpy as np
import jax
import jax.numpy as jnp
from jax import lax
from jax.experimental import pallas as pl
from jax.experimental.pallas import tpu as pltpu
from jax.experimental.pallas import tpu_sc as plsc

L_LEVELS = 16
F_FEAT = 2
T_SIZE = 2 ** 19
B_PTS = 262144
APP_DIM = 27

P2 = np.int32(-1640531535)
P3 = np.int32(805459861)

NW = 32
PTS_PER_W = B_PTS // NW
BLK = 1024
NCHUNK = BLK // 16
NBLOCKS = PTS_PER_W // BLK
PIPE = 4

DL = 5
D_G = [3, 5, 9, 17, 33]
D_OFF = [0, 27, 152, 881, 5794]
D_TOT = 41731
NSTREAM = L_LEVELS - DL
CW = NSTREAM * 128
D_PAD = 8 * (4 * CW)

def _dense_idx():
    parts = []
    for l in range(DL):
        g = np.uint32(D_G[l])
        xs = np.arange(g, dtype=np.uint32)
        hy = xs * np.uint32(2654435761)
        hz = xs * np.uint32(805459861)
        h = (xs[:, None, None] ^ hy[None, :, None] ^ hz[None, None, :]) \
            & np.uint32(T_SIZE - 1)
        parts.append((np.int64(l * T_SIZE) + h.reshape(-1)).astype(np.int32))
    flat = np.concatenate(parts)
    return np.concatenate([flat, np.zeros(D_PAD - flat.size, np.int32)])

DIDX = _dense_idx()

_MASK = np.int32(T_SIZE - 1)


def _encode_body(cx_hbm, cy_hbm, cz_hbm, cd_hbm, tpk_hbm, dense_hbm, out_hbm,
                 cx_v, cy_v, cz_v, cd_v, idx_v, rows_v, feat_v, dense_v,
                 gsem):
    nc = 2
    wid = lax.axis_index("s") * nc + lax.axis_index("c")
    lanes = lax.iota(jnp.int32, 16)

    def load_xyz(o):
        invd = cd_v[pl.ds(o, 16)]
        return (cx_v[pl.ds(o, 16)] * invd,
                cy_v[pl.ds(o, 16)] * invd,
                cz_v[pl.ds(o, 16)] * invd)

    def fire(ci, par):
        x0, x1, x2 = load_xyz(ci * 16)
        for l in range(DL, L_LEVELS):
            res = np.float32(2.0 * (2.0 ** l))
            px = x0 * res
            py = x1 * res
            pz = x2 * res
            ix = px.astype(jnp.int32)
            iy = py.astype(jnp.int32)
            iz = pz.astype(jnp.int32)
            hx0 = ix
            hx1 = ix + np.int32(1)
            hy0 = iy * P2
            hy1 = hy0 + P2
            hz0 = iz * P3
            hz1 = hz0 + P3
            base_l = np.int32(l * T_SIZE)
            corner = 0
            for hx in (hx0, hx1):
                for hy in (hy0, hy1):
                    for hz in (hz0, hz1):
                        idx = ((hx ^ hy ^ hz) & _MASK) + base_l
                        idx_v[pl.ds(par * np.int32(CW) + np.int32((l - DL) * 128 + corner * 16), 16)] = idx
                        corner += 1
        pltpu.async_copy(tpk_hbm.at[idx_v.at[pl.ds(par * np.int32(CW), CW)]],
                         rows_v.at[pl.ds(par * np.int32(CW), CW)], gsem)

    def drain_and_interp(ci, par):
        pltpu.make_async_copy(tpk_hbm.at[idx_v.at[pl.ds(par * np.int32(CW), CW)]],
                              rows_v.at[pl.ds(par * np.int32(CW), CW)],
                              gsem).wait()

        o = ci * 16
        x0, x1, x2 = load_xyz(o)
        pbase_i = (o + lanes) * np.int32(2 * L_LEVELS)
        for l in range(L_LEVELS):
            res = np.float32(2.0 * (2.0 ** l))
            px = x0 * res
            py = x1 * res
            pz = x2 * res
            ix = px.astype(jnp.int32)
            iy = py.astype(jnp.int32)
            iz = pz.astype(jnp.int32)
            w0 = px - ix.astype(jnp.float32)
            w1 = py - iy.astype(jnp.float32)
            w2 = pz - iz.astype(jnp.float32)
            u0 = np.float32(1.0) - w0
            u1 = np.float32(1.0) - w1
            u2 = np.float32(1.0) - w2
            acc0 = jnp.zeros((16,), jnp.float32)
            acc1 = jnp.zeros((16,), jnp.float32)
            if l < DL:
                g = D_G[l]
                gg = g * g
                e0 = ix * np.int32(gg) + iy * np.int32(g) + iz + np.int32(D_OFF[l])
            else:
                row = par * np.int32(CW) + np.int32((l - DL) * 128)
            corner = 0
            for ci_x, wxv in ((0, u0), (1, w0)):
                for ci_y, wyv in ((0, u1), (1, w1)):
                    for ci_z, wzv in ((0, u2), (1, w2)):
                        ww = (wxv * wyv) * wzv
                        if l < DL:
                            off = ci_x * gg + ci_y * g + ci_z
                            v = plsc.load_gather(dense_v, [e0 + np.int32(off)])
                        else:
                            v = rows_v[pl.ds(row + np.int32(corner * 16), 16)]
                        f0 = plsc.bitcast(v & np.int32(-65536), jnp.float32)
                        f1 = plsc.bitcast(v << np.int32(16), jnp.float32)
                        acc0 = acc0 + f0 * ww
                        acc1 = acc1 + f1 * ww
                        corner += 1
            plsc.store_scatter(feat_v, [pbase_i + np.int32(2 * l)], acc0)
            plsc.store_scatter(feat_v, [pbase_i + np.int32(2 * l + 1)], acc1)

    pltpu.sync_copy(dense_hbm, dense_v)

    def block_body(b, carry):
        pbase = wid * PTS_PER_W + b * BLK
        pltpu.sync_copy(cx_hbm.at[pl.ds(pbase, BLK)], cx_v)
        pltpu.sync_copy(cy_hbm.at[pl.ds(pbase, BLK)], cy_v)
        pltpu.sync_copy(cz_hbm.at[pl.ds(pbase, BLK)], cz_v)
        pltpu.sync_copy(cd_hbm.at[pl.ds(pbase, BLK)], cd_v)

        def chunk_body(ci, carry2):
            par = lax.rem(ci, np.int32(PIPE))

            @pl.when(ci >= PIPE)
            def _():
                drain_and_interp(ci - PIPE, par)

            @pl.when(ci < NCHUNK)
            def _():
                fire(ci, par)

            return carry2

        lax.fori_loop(0, NCHUNK + PIPE, chunk_body, 0)
        pltpu.sync_copy(feat_v, out_hbm.at[pl.ds(pbase * np.int32(2 * L_LEVELS),
                                                 BLK * 2 * L_LEVELS)])
        return carry

    lax.fori_loop(0, NBLOCKS, block_body, 0)


@jax.jit
def _encode(cx, cy, cz, cd, tpk, dense):
    mesh = plsc.VectorSubcoreMesh(core_axis_name="c", subcore_axis_name="s")
    fn = functools.partial(
        pl.kernel,
        mesh=mesh,
        compiler_params=pltpu.CompilerParams(needs_layout_passes=False),
        out_type=jax.ShapeDtypeStruct((B_PTS * 2 * L_LEVELS,), jnp.float32),
        scratch_types=[
            pltpu.VMEM((BLK,), jnp.float32),
            pltpu.VMEM((BLK,), jnp.float32),
            pltpu.VMEM((BLK,), jnp.float32),
            pltpu.VMEM((BLK,), jnp.float32),
            pltpu.VMEM((PIPE * CW,), jnp.int32),
            pltpu.VMEM((PIPE * CW,), jnp.int32),
            pltpu.VMEM((BLK * 2 * L_LEVELS,), jnp.float32),
            pltpu.VMEM((D_PAD,), jnp.int32),
            pltpu.SemaphoreType.DMA,
        ],
    )(_encode_body)
    return fn(cx, cy, cz, cd, tpk, dense).reshape(B_PTS, 2 * L_LEVELS)


def _mlp_body(feat_ref, w1_ref, w2_ref, w3_ref, sig_ref, app_ref):
    f = feat_ref[...]
    h = jnp.maximum(jnp.dot(f, w1_ref[...], preferred_element_type=jnp.float32), 0.0)
    h = jnp.maximum(jnp.dot(h, w2_ref[...], preferred_element_type=jnp.float32), 0.0)
    out = jnp.dot(h, w3_ref[...], preferred_element_type=jnp.float32)
    sig_ref[...] = out[:, :1]
    app_ref[...] = out[:, 1:1 + APP_DIM]


@jax.jit
def _mlp(feats, W1, W2, W3):
    BM = 2048
    return pl.pallas_call(
        _mlp_body,
        grid=(B_PTS // BM,),
        in_specs=[
            pl.BlockSpec((BM, 2 * L_LEVELS), lambda i: (i, 0)),
            pl.BlockSpec((2 * L_LEVELS, 128), lambda i: (0, 0)),
            pl.BlockSpec((128, 128), lambda i: (0, 0)),
            pl.BlockSpec((128, 32), lambda i: (0, 0)),
        ],
        out_specs=[
            pl.BlockSpec((BM, 1), lambda i: (i, 0)),
            pl.BlockSpec((BM, APP_DIM), lambda i: (i, 0)),
        ],
        out_shape=[
            jax.ShapeDtypeStruct((B_PTS, 1), jnp.float32),
            jax.ShapeDtypeStruct((B_PTS, APP_DIM), jnp.float32),
        ],
    )(feats, W1, W2, W3)


def kernel(xyz_env_normed, table, W1d, W2d, W3d, W1r, W2r, W3r):
    cx = xyz_env_normed[:, 0]
    cy = xyz_env_normed[:, 1]
    cz = xyz_env_normed[:, 2]
    cd = xyz_env_normed[:, 3]
    bits = jax.lax.bitcast_convert_type(
        table.astype(jnp.bfloat16), jnp.uint16).astype(jnp.uint32)
    tpk = jax.lax.bitcast_convert_type(
        (bits[:, :, 0] << jnp.uint32(16)) | bits[:, :, 1],
        jnp.int32).reshape(L_LEVELS * T_SIZE)

    dense = jnp.take(tpk, jnp.asarray(DIDX))

    feats = _encode(cx, cy, cz, cd, tpk, dense)

    Z = jnp.zeros((64, 64), jnp.float32)
    W1 = jnp.concatenate([W1d, W1r], axis=1)
    W2 = jnp.concatenate(
        [jnp.concatenate([W2d, Z], axis=1),
         jnp.concatenate([Z, W2r], axis=1)], axis=0)
    W3 = jnp.zeros((128, 32), jnp.float32)
    W3 = W3.at[:64, 0:1].set(W3d)
    W3 = W3.at[64:, 1:1 + APP_DIM].set(W3r)

    sig, app_feat = _mlp(feats, W1, W2, W3)
    return (sig[:, 0], app_feat)
